# baseline (speedup 1.0000x reference)
import functools

import jax
import jax.numpy as jnp
from jax import lax
from jax.experimental import pallas as pl
from jax.experimental.pallas import tpu as pltpu
from jax.experimental.pallas import tpu_sc as plsc

_B = 16384
_D = 64
_DECAY = 0.0001
_L = 16
_NC = 2
_NS = 16
_NW = _NC * _NS
_BPW = _B // _NW
_CK = 16
_NCK = _BPW // _CK


def _sc_body(u2, p2, n2, utab, itab, part_out, sq_out,
             idx_u, idx_p, idx_n, rows_u, rows_p, rows_n, part_buf, sq_buf,
             drain_buf, sem_a, sem_b):
    c = lax.axis_index("c")
    s = lax.axis_index("s")
    w = s * _NC + c

    pltpu.sync_copy(u2.at[w], idx_u)
    pltpu.sync_copy(p2.at[w], idx_p)
    pltpu.sync_copy(n2.at[w], idx_n)

    def issue(k, phase, sem):
        sl = pl.ds(k * _CK, _CK)
        iu_v = idx_u[sl]
        ip_v = idx_p[sl]
        iq_v = idx_n[sl]
        bu_v = (iu_v >> 3) * 8
        bp_v = (ip_v >> 3) * 8
        bq_v = (iq_v >> 3) * 8
        for ii in range(_CK):
            dst = pl.ds(phase * 128 + ii * 8, 8)
            bu = pl.multiple_of(bu_v[ii], 8)
            bp = pl.multiple_of(bp_v[ii], 8)
            bq = pl.multiple_of(bq_v[ii], 8)
            pltpu.async_copy(utab.at[pl.ds(bu, 8)], rows_u.at[dst], sem)
            pltpu.async_copy(itab.at[pl.ds(bp, 8)], rows_p.at[dst], sem)
            pltpu.async_copy(itab.at[pl.ds(bq, 8)], rows_n.at[dst], sem)

    def drain(sem):
        pltpu.make_async_copy(utab.at[pl.ds(0, 3 * _CK * 8)], drain_buf, sem).wait()

    def compute(k, phase, sq):
        sl = pl.ds(k * _CK, _CK)
        su_v = idx_u[sl] & 7
        sp_v = idx_p[sl] & 7
        sq_v = idx_n[sl] & 7
        for ii in range(_CK):
            ru = phase * 128 + ii * 8 + su_v[ii]
            rp = phase * 128 + ii * 8 + sp_v[ii]
            rn = phase * 128 + ii * 8 + sq_v[ii]
            acc = jnp.zeros((_L,), jnp.float32)
            for ch in range(_D // _L):
                csl = pl.ds(ch * _L, _L)
                u = rows_u[ru, csl]
                p = rows_p[rp, csl]
                n = rows_n[rn, csl]
                acc = acc + u * (p - n)
                sq = sq + u * u + p * p + n * n
            part_buf[pl.ds((k * _CK + ii) * _L, _L)] = acc
        return sq

    issue(0, 0, sem_a)

    def body(kk, sq):
        issue(2 * kk + 1, 1, sem_b)
        drain(sem_a)
        sq = compute(2 * kk, 0, sq)
        issue(2 * kk + 2, 0, sem_a)
        drain(sem_b)
        return compute(2 * kk + 1, 1, sq)

    sq = lax.fori_loop(0, _NCK // 2 - 1, body, jnp.zeros((_L,), jnp.float32))
    issue(_NCK - 1, 1, sem_b)
    drain(sem_a)
    sq = compute(_NCK - 2, 0, sq)
    drain(sem_b)
    sq = compute(_NCK - 1, 1, sq)

    sq_buf[...] = sq
    pltpu.sync_copy(part_buf, part_out.at[w])
    pltpu.sync_copy(sq_buf, sq_out.at[w])


_sc_kernel = functools.partial(
    pl.kernel,
    out_type=[
        jax.ShapeDtypeStruct((_NW, _BPW * _L), jnp.float32),
        jax.ShapeDtypeStruct((_NW, _L), jnp.float32),
    ],
    mesh=plsc.VectorSubcoreMesh(core_axis_name="c", subcore_axis_name="s"),
    scratch_types=[
        pltpu.VMEM((_BPW,), jnp.int32),
        pltpu.VMEM((_BPW,), jnp.int32),
        pltpu.VMEM((_BPW,), jnp.int32),
        pltpu.VMEM((256, _D), jnp.float32),
        pltpu.VMEM((256, _D), jnp.float32),
        pltpu.VMEM((256, _D), jnp.float32),
        pltpu.VMEM((_BPW * _L,), jnp.float32),
        pltpu.VMEM((_L,), jnp.float32),
        pltpu.VMEM((3 * _CK * 8, _D), jnp.float32),
        pltpu.SemaphoreType.DMA,
        pltpu.SemaphoreType.DMA,
    ],
)(_sc_body)


def _tc_body(part_ref, sq_ref, out_ref):
    x = part_ref[...]
    scores = jnp.sum(x, axis=1, keepdims=True)
    ls = jnp.minimum(scores, 0.0) - jnp.log1p(jnp.exp(-jnp.abs(scores)))
    mf = -jnp.sum(ls) / _B
    reg = 0.5 * _DECAY * jnp.sum(sq_ref[...])
    out_ref[...] = jnp.full((1, 1), mf + reg, jnp.float32)


def kernel(users, pos_items, neg_items, user_embedding, item_embedding):
    u2 = users.reshape(_NW, _BPW)
    p2 = pos_items.reshape(_NW, _BPW)
    n2 = neg_items.reshape(_NW, _BPW)
    part, sq = _sc_kernel(u2, p2, n2, user_embedding, item_embedding)
    out = pl.pallas_call(
        _tc_body,
        out_shape=jax.ShapeDtypeStruct((1, 1), jnp.float32),
    )(part.reshape(_B, _L), sq)
    prime_u = jnp.sum(jnp.take(user_embedding, users[:8], axis=0))
    prime_i = jnp.sum(jnp.take(item_embedding, pos_items[:8], axis=0))
    zero = jnp.minimum(jnp.abs(prime_u) + jnp.abs(prime_i), 0.0)
    return out[0, 0] + zero


# speedup vs baseline: 1.0020x; 1.0020x over previous
import functools

import jax
import jax.numpy as jnp
from jax import lax
from jax.experimental import pallas as pl
from jax.experimental.pallas import tpu as pltpu
from jax.experimental.pallas import tpu_sc as plsc

_B = 16384
_D = 64
_DECAY = 0.0001
_L = 16
_NC = 2
_NS = 16
_NW = _NC * _NS
_BPW = _B // _NW
_CK = 16
_NCK = _BPW // _CK


def _sc_body(u2, p2, n2, utab, itab, part_out, sq_out,
             idx_u, idx_p, idx_n, rows_u, rows_p, rows_n, part_buf, sq_buf,
             drain_buf, sem_a, sem_b):
    c = lax.axis_index("c")
    s = lax.axis_index("s")
    w = s * _NC + c

    pltpu.sync_copy(u2.at[w], idx_u)
    pltpu.sync_copy(p2.at[w], idx_p)
    pltpu.sync_copy(n2.at[w], idx_n)

    def issue(k, phase, sem):
        sl = pl.ds(k * _CK, _CK)
        iu_v = idx_u[sl]
        ip_v = idx_p[sl]
        iq_v = idx_n[sl]
        bu_v = (iu_v >> 3) * 8
        bp_v = (ip_v >> 3) * 8
        bq_v = (iq_v >> 3) * 8
        for ii in range(_CK):
            dst = pl.ds(phase * 128 + ii * 8, 8)
            bu = pl.multiple_of(bu_v[ii], 8)
            bp = pl.multiple_of(bp_v[ii], 8)
            bq = pl.multiple_of(bq_v[ii], 8)
            pltpu.async_copy(utab.at[pl.ds(bu, 8)], rows_u.at[dst], sem)
            pltpu.async_copy(itab.at[pl.ds(bp, 8)], rows_p.at[dst], sem)
            pltpu.async_copy(itab.at[pl.ds(bq, 8)], rows_n.at[dst], sem)

    def drain(sem):
        pltpu.make_async_copy(utab.at[pl.ds(0, 3 * _CK * 8)], drain_buf, sem).wait()

    def compute(k, phase, sq):
        sl = pl.ds(k * _CK, _CK)
        su_v = idx_u[sl] & 7
        sp_v = idx_p[sl] & 7
        sq_v = idx_n[sl] & 7
        for ii in range(_CK):
            ru = phase * 128 + ii * 8 + su_v[ii]
            rp = phase * 128 + ii * 8 + sp_v[ii]
            rn = phase * 128 + ii * 8 + sq_v[ii]
            acc = jnp.zeros((_L,), jnp.float32)
            for ch in range(_D // _L):
                csl = pl.ds(ch * _L, _L)
                u = rows_u[ru, csl]
                p = rows_p[rp, csl]
                n = rows_n[rn, csl]
                acc = acc + u * (p - n)
                sq = sq + u * u + p * p + n * n
            part_buf[pl.ds((k * _CK + ii) * _L, _L)] = acc
        return sq

    issue(0, 0, sem_a)

    def body(kk, sq):
        issue(2 * kk + 1, 1, sem_b)
        drain(sem_a)
        sq = compute(2 * kk, 0, sq)
        issue(2 * kk + 2, 0, sem_a)
        drain(sem_b)
        return compute(2 * kk + 1, 1, sq)

    sq = lax.fori_loop(0, _NCK // 2 - 1, body, jnp.zeros((_L,), jnp.float32))
    issue(_NCK - 1, 1, sem_b)
    drain(sem_a)
    sq = compute(_NCK - 2, 0, sq)
    drain(sem_b)
    sq = compute(_NCK - 1, 1, sq)

    sq_buf[...] = sq
    pltpu.sync_copy(part_buf, part_out.at[w])
    pltpu.sync_copy(sq_buf, sq_out.at[w])


_sc_kernel = functools.partial(
    pl.kernel,
    out_type=[
        jax.ShapeDtypeStruct((_NW, _BPW * _L), jnp.float32),
        jax.ShapeDtypeStruct((_NW, _L), jnp.float32),
    ],
    mesh=plsc.VectorSubcoreMesh(core_axis_name="c", subcore_axis_name="s"),
    scratch_types=[
        pltpu.VMEM((_BPW,), jnp.int32),
        pltpu.VMEM((_BPW,), jnp.int32),
        pltpu.VMEM((_BPW,), jnp.int32),
        pltpu.VMEM((256, _D), jnp.float32),
        pltpu.VMEM((256, _D), jnp.float32),
        pltpu.VMEM((256, _D), jnp.float32),
        pltpu.VMEM((_BPW * _L,), jnp.float32),
        pltpu.VMEM((_L,), jnp.float32),
        pltpu.VMEM((3 * _CK * 8, _D), jnp.float32),
        pltpu.SemaphoreType.DMA,
        pltpu.SemaphoreType.DMA,
    ],
)(_sc_body)


def _tc_body(part_ref, sq_ref, out_ref):
    x = part_ref[...]
    scores = jnp.sum(x, axis=1, keepdims=True)
    ls = jnp.minimum(scores, 0.0) - jnp.log1p(jnp.exp(-jnp.abs(scores)))
    mf = -jnp.sum(ls) / _B
    reg = 0.5 * _DECAY * jnp.sum(sq_ref[...])
    out_ref[...] = jnp.full((1, 1), mf + reg, jnp.float32)


def kernel(users, pos_items, neg_items, user_embedding, item_embedding):
    u2 = users.reshape(_NW, _BPW)
    p2 = pos_items.reshape(_NW, _BPW)
    n2 = neg_items.reshape(_NW, _BPW)
    part, sq = _sc_kernel(u2, p2, n2, user_embedding, item_embedding)
    out = pl.pallas_call(
        _tc_body,
        out_shape=jax.ShapeDtypeStruct((1, 1), jnp.float32),
    )(part.reshape(_B, _L), sq)
    return out[0, 0]
